# baseline (speedup 1.0000x reference)
import jax
import jax.numpy as jnp
from jax.experimental import pallas as pl
from jax.experimental.pallas import tpu as pltpu

N_B, N_A, N_D, N_I = 50000, 50000, 10000, 10000
IN_DIM, H = 128, 128
E = 500000

ROW_BLK = 512


def _lnproj_body(x_ref, g_ref, b_ref, w_ref, pb_ref, o_ref):
    x = x_ref[...]
    mu = jnp.mean(x, axis=-1, keepdims=True)
    var = jnp.mean((x - mu) ** 2, axis=-1, keepdims=True)
    xn = ((x - mu) / jnp.sqrt(var + 1e-5)) * g_ref[...] + b_ref[...]
    o_ref[...] = jnp.dot(xn, w_ref[...], preferred_element_type=jnp.float32) + pb_ref[...]


def _lnproj(x, g, b, W, pb):
    n = x.shape[0]
    n_pad = (n + ROW_BLK - 1) // ROW_BLK * ROW_BLK
    if n_pad != n:
        x = jnp.pad(x, ((0, n_pad - n), (0, 0)))
    out = pl.pallas_call(
        _lnproj_body,
        grid=(n_pad // ROW_BLK,),
        in_specs=[
            pl.BlockSpec((ROW_BLK, IN_DIM), lambda i: (i, 0)),
            pl.BlockSpec((1, IN_DIM), lambda i: (0, 0)),
            pl.BlockSpec((1, IN_DIM), lambda i: (0, 0)),
            pl.BlockSpec((IN_DIM, H), lambda i: (0, 0)),
            pl.BlockSpec((1, H), lambda i: (0, 0)),
        ],
        out_specs=pl.BlockSpec((ROW_BLK, H), lambda i: (i, 0)),
        out_shape=jax.ShapeDtypeStruct((n_pad, H), jnp.float32),
    )(x, g.reshape(1, -1), b.reshape(1, -1), W, pb.reshape(1, -1))
    return out[:n]


def kernel(bidder_x, ei_ba, ei_ab, ei_bd, ei_db, ei_bi, ei_ib, ln_g, ln_b, proj_W, proj_b, auction_emb, device_emb, ip_emb, Wl1, bl1, Wr1, Wl2, bl2, Wr2, hW1, hb1, hW2, hb2):
    def conv(k, x_src, x_dst, ei, n_dst, Wl, bl, Wr):
        src, dst = ei[0], ei[1]
        s = jax.ops.segment_sum(x_src[src], dst, num_segments=n_dst)
        c = jax.ops.segment_sum(jnp.ones((ei.shape[1],), jnp.float32), dst, num_segments=n_dst)
        mean = s / jnp.maximum(c, 1.0)[:, None]
        return mean @ Wl[k] + bl[k] + x_dst @ Wr[k]

    def layer(xb, xa, xd, xi, Wl, bl, Wr):
        a = conv(0, xb, xa, ei_ba, N_A, Wl, bl, Wr)
        b = (conv(1, xa, xb, ei_ab, N_B, Wl, bl, Wr)
             + conv(3, xd, xb, ei_db, N_B, Wl, bl, Wr)
             + conv(5, xi, xb, ei_ib, N_B, Wl, bl, Wr))
        d = conv(2, xb, xd, ei_bd, N_D, Wl, bl, Wr)
        i = conv(4, xb, xi, ei_bi, N_I, Wl, bl, Wr)
        return jax.nn.relu(b), jax.nn.relu(a), jax.nn.relu(d), jax.nn.relu(i)

    xb = _lnproj(bidder_x, ln_g, ln_b, proj_W, proj_b)
    xb, xa, xd, xi = layer(xb, auction_emb, device_emb, ip_emb, Wl1, bl1, Wr1)
    xb, xa, xd, xi = layer(xb, xa, xd, xi, Wl2, bl2, Wr2)
    h = jax.nn.gelu(xb @ hW1 + hb1, approximate=False)
    return (h @ hW2 + hb2)[:, 0]


# speedup vs baseline: 1.7804x; 1.7804x over previous
import dataclasses
import functools

import jax
import jax.numpy as jnp
from jax import lax
from jax.experimental import pallas as pl
from jax.experimental.pallas import tpu as pltpu
from jax.experimental.pallas import tpu_sc as plsc

N_B, N_A, N_D, N_I = 50000, 50000, 10000, 10000
IN_DIM, H = 128, 128
E = 500000

NPAD_BIG = 50688
NPAD_SMALL = 10240
PART_BIG = 4224
PART_SMALL = 2560
PART_MAX = PART_BIG

E_PAD = 507904
SUB_E = E_PAD // 16
N_CHUNK = SUB_E // 1024
CAP_ROWS = (SUB_E + 128 + 127) // 128 + 1

ROW_BLK = 512

_MESH = plsc.VectorSubcoreMesh(core_axis_name="c", subcore_axis_name="s")
_CP = pltpu.CompilerParams()
if "needs_layout_passes" in pltpu.CompilerParams.__dataclass_fields__:
    _CP = dataclasses.replace(_CP, needs_layout_passes=False)


def _zero_stripes(sub, part, zbuf, bufs):
    stripe = part // 16
    zbase = sub * stripe
    nfull, rem = stripe // 64, stripe % 64
    zb = [(b * 64, 64) for b in range(nfull)]
    if rem:
        zb.append((nfull * 64, rem))
    for r0, bs in zb:
        for buf in bufs:
            pltpu.sync_copy(zbuf.at[pl.ds(0, bs)], buf.at[pl.ds(zbase + r0, bs)])

    @pl.when(sub < 4)
    def _():
        for buf in bufs:
            pltpu.sync_copy(zbuf.at[pl.ds(0, 8)], buf.at[pl.ds(part + sub * 8, 8)])

    return zb


def _compact(lo, part, src_hbm, dst_hbm, sidx, didx, csrc, cdst, sub, with_src):
    ebase = sub * SUB_E

    def chunk_body(ch, off):
        cb = ebase + ch * 1024
        if with_src:
            pltpu.sync_copy(src_hbm.at[pl.ds(cb, 1024)], sidx)
        pltpu.sync_copy(dst_hbm.at[pl.ds(cb, 1024)], didx)

        def vbody(v, off):
            d = didx[pl.ds(v * 16, 16)]
            m = (d >= lo) & (d < lo + part)
            mi = m.astype(jnp.int32)
            pc = plsc.cumsum(mi)
            pos = off + pc - 1
            ri, ci = pos >> 7, pos & 127
            if with_src:
                s = sidx[pl.ds(v * 16, 16)]
                plsc.store_scatter(csrc, [ri, ci], s, mask=m)
            plsc.store_scatter(cdst, [ri, ci], d - lo, mask=m)
            return off + pc[15]

        return lax.fori_loop(0, 64, vbody, off)

    n_sel = lax.fori_loop(0, N_CHUNK, chunk_body, jnp.int32(0))

    io = lax.iota(jnp.int32, 16)
    for t in range(8):
        pidx = n_sel + t * 16 + io
        ri, ci = pidx >> 7, pidx & 127
        if with_src:
            plsc.store_scatter(csrc, [ri, ci], io + t * 16)
        plsc.store_scatter(cdst, [ri, ci], part + ((io + t * 16) & 31))
    return (n_sel + 127) >> 7


def _emit_segsum(core, sub, part, parts_per_core,
                 y_hbm, src_hbm, dst_hbm, out_hbm,
                 sidx, didx, csrc, cdst, rows, zbuf, acc):
    stripe = part // 16

    @pl.loop(0, parts_per_core)
    def _(p_local):
        p = core * parts_per_core + p_local
        lo = p * part

        zb = _zero_stripes(sub, part, zbuf, [acc])
        plsc.subcore_barrier()

        nb = _compact(lo, part, src_hbm, dst_hbm, sidx, didx, csrc, cdst, sub, True)

        def bbody(j, carry):
            pltpu.sync_copy(y_hbm.at[csrc.at[j]], rows)
            pltpu.sync_copy(rows, acc.at[cdst.at[j]], add=True)
            return carry

        lax.fori_loop(0, nb, bbody, jnp.int32(0))
        plsc.subcore_barrier()

        fbase = sub * stripe
        for r0, bs in zb:
            pltpu.sync_copy(acc.at[pl.ds(fbase + r0, bs)],
                            out_hbm.at[pl.ds(lo + fbase + r0, bs)])
        plsc.subcore_barrier()


def _emit_invcount(core, sub, part, parts_per_core,
                   dst_hbm, out_hbm,
                   sidx, didx, csrc, cdst, ones, zbuf, blk, acc):
    stripe = part // 16

    @pl.loop(0, parts_per_core)
    def _(p_local):
        p = core * parts_per_core + p_local
        lo = p * part

        zb = _zero_stripes(sub, part, zbuf, [acc])
        plsc.subcore_barrier()

        nb = _compact(lo, part, None, dst_hbm, sidx, didx, csrc, cdst, sub, False)

        def bbody(j, carry):
            pltpu.sync_copy(ones, acc.at[cdst.at[j]], add=True)
            return carry

        lax.fori_loop(0, nb, bbody, jnp.int32(0))
        plsc.subcore_barrier()

        fbase = sub * stripe
        for r0, bs in zb:
            pltpu.sync_copy(acc.at[pl.ds(fbase + r0, bs)], blk.at[pl.ds(0, bs)])

            @pl.loop(0, bs)
            def _(r):
                for c in range(8):
                    slc = (r, pl.ds(c * 16, 16))
                    blk[slc] = 1.0 / jnp.maximum(blk[slc], 1.0)

            pltpu.sync_copy(blk.at[pl.ds(0, bs)],
                            out_hbm.at[pl.ds(lo + fbase + r0, bs)])
        plsc.subcore_barrier()


def _layer_segsums_body(xb, xa, xd, xi,
                        sba, dba, sab, dab, sdb, ddb, sib, dib, sbd, dbd, sbi, dbi,
                        m_ba, m_ab, m_db, m_ib, m_bd, m_bi,
                        sidx, didx, csrc, cdst, rows, zbuf, acc):
    core = lax.axis_index("c")
    sub = lax.axis_index("s")

    @pl.loop(0, 64)
    def _(i):
        for c in range(8):
            zbuf[i, pl.ds(c * 16, 16)] = jnp.zeros((16,), jnp.float32)

    convs = [
        (xb, sba, dba, m_ba, PART_BIG, 6),
        (xa, sab, dab, m_ab, PART_BIG, 6),
        (xd, sdb, ddb, m_db, PART_BIG, 6),
        (xi, sib, dib, m_ib, PART_BIG, 6),
        (xb, sbd, dbd, m_bd, PART_SMALL, 2),
        (xb, sbi, dbi, m_bi, PART_SMALL, 2),
    ]
    for y, s_e, d_e, out, prt, ppc in convs:
        _emit_segsum(core, sub, prt, ppc, y, s_e, d_e, out,
                     sidx, didx, csrc, cdst, rows, zbuf, acc)


def _counts_body(dba, dab, ddb, dib, dbd, dbi,
                 ic_ba, ic_ab, ic_db, ic_ib, ic_bd, ic_bi,
                 sidx, didx, csrc, cdst, ones, zbuf, blk, acc):
    core = lax.axis_index("c")
    sub = lax.axis_index("s")

    @pl.loop(0, 64)
    def _(i):
        for c in range(8):
            zbuf[i, pl.ds(c * 16, 16)] = jnp.zeros((16,), jnp.float32)

    @pl.loop(0, 128)
    def _(i):
        for c in range(8):
            ones[i, pl.ds(c * 16, 16)] = jnp.ones((16,), jnp.float32)

    cfgs = [
        (dba, ic_ba, PART_BIG, 6),
        (dab, ic_ab, PART_BIG, 6),
        (ddb, ic_db, PART_BIG, 6),
        (dib, ic_ib, PART_BIG, 6),
        (dbd, ic_bd, PART_SMALL, 2),
        (dbi, ic_bi, PART_SMALL, 2),
    ]
    for d_e, out, prt, ppc in cfgs:
        _emit_invcount(core, sub, prt, ppc, d_e, out,
                       sidx, didx, csrc, cdst, ones, zbuf, blk, acc)


_BIG_T = jax.ShapeDtypeStruct((NPAD_BIG, H), jnp.float32)
_SMALL_T = jax.ShapeDtypeStruct((NPAD_SMALL, H), jnp.float32)

_layer_segsums = pl.kernel(
    _layer_segsums_body,
    out_type=[_BIG_T, _BIG_T, _BIG_T, _BIG_T, _SMALL_T, _SMALL_T],
    mesh=_MESH,
    compiler_params=_CP,
    scratch_types=[
        pltpu.VMEM((1024,), jnp.int32),
        pltpu.VMEM((1024,), jnp.int32),
        pltpu.VMEM((CAP_ROWS, 128), jnp.int32),
        pltpu.VMEM((CAP_ROWS, 128), jnp.int32),
        pltpu.VMEM((128, 128), jnp.float32),
        pltpu.VMEM((64, 128), jnp.float32),
        pltpu.VMEM_SHARED((PART_MAX + 32, 128), jnp.float32),
    ],
)

_edge_counts = pl.kernel(
    _counts_body,
    out_type=[_BIG_T, _BIG_T, _BIG_T, _BIG_T, _SMALL_T, _SMALL_T],
    mesh=_MESH,
    compiler_params=_CP,
    scratch_types=[
        pltpu.VMEM((1024,), jnp.int32),
        pltpu.VMEM((1024,), jnp.int32),
        pltpu.VMEM((CAP_ROWS, 128), jnp.int32),
        pltpu.VMEM((CAP_ROWS, 128), jnp.int32),
        pltpu.VMEM((128, 128), jnp.float32),
        pltpu.VMEM((64, 128), jnp.float32),
        pltpu.VMEM((64, 128), jnp.float32),
        pltpu.VMEM_SHARED((PART_MAX + 32, 128), jnp.float32),
    ],
)


def _pad_edges(ei, n_dst_pad):
    pad = E_PAD - E
    srcp = jnp.concatenate([ei[0], jnp.zeros((pad,), jnp.int32)])
    dstp = jnp.concatenate([ei[1], jnp.full((pad,), n_dst_pad, jnp.int32)])
    return srcp, dstp


def _lnproj_body(x_ref, g_ref, b_ref, w_ref, pb_ref, o_ref):
    x = x_ref[...]
    mu = jnp.mean(x, axis=-1, keepdims=True)
    var = jnp.mean((x - mu) ** 2, axis=-1, keepdims=True)
    xn = ((x - mu) / jnp.sqrt(var + 1e-5)) * g_ref[...] + b_ref[...]
    o_ref[...] = jnp.dot(xn, w_ref[...], preferred_element_type=jnp.float32) + pb_ref[...]


def _lnproj(x, g, b, W, pb, n_pad):
    x = jnp.pad(x, ((0, n_pad - x.shape[0]), (0, 0)))
    return pl.pallas_call(
        _lnproj_body,
        grid=(n_pad // ROW_BLK,),
        in_specs=[
            pl.BlockSpec((ROW_BLK, IN_DIM), lambda i: (i, 0)),
            pl.BlockSpec((1, IN_DIM), lambda i: (0, 0)),
            pl.BlockSpec((1, IN_DIM), lambda i: (0, 0)),
            pl.BlockSpec((IN_DIM, H), lambda i: (0, 0)),
            pl.BlockSpec((1, H), lambda i: (0, 0)),
        ],
        out_specs=pl.BlockSpec((ROW_BLK, H), lambda i: (i, 0)),
        out_shape=jax.ShapeDtypeStruct((n_pad, H), jnp.float32),
    )(x, g.reshape(1, -1), b.reshape(1, -1), W, pb.reshape(1, -1))


def _combine_body(m, refs):
    sums = refs[:m]
    ics = refs[m:2 * m]
    xd_ref, wl_ref, wr_ref, b_ref, o_ref = refs[2 * m:]
    x = jnp.concatenate(
        [s[...] * ic[...] for s, ic in zip(sums, ics)], axis=-1)
    acc = jnp.dot(x, wl_ref[...], preferred_element_type=jnp.float32)
    xd = xd_ref[...]
    for k in range(m):
        acc = acc + jnp.dot(xd, wr_ref[k], preferred_element_type=jnp.float32)
    o_ref[...] = jax.nn.relu(acc + b_ref[...])


def _combine(sums, ics, x_dst, Wl_cat, Wr_stack, bias):
    m = len(sums)
    n = x_dst.shape[0]
    body = lambda *refs: _combine_body(m, refs)
    return pl.pallas_call(
        body,
        grid=(n // ROW_BLK,),
        in_specs=[pl.BlockSpec((ROW_BLK, H), lambda i: (i, 0))
                  for _ in range(2 * m + 1)]
        + [
            pl.BlockSpec((m * H, H), lambda i: (0, 0)),
            pl.BlockSpec((m, H, H), lambda i: (0, 0, 0)),
            pl.BlockSpec((1, H), lambda i: (0, 0)),
        ],
        out_specs=pl.BlockSpec((ROW_BLK, H), lambda i: (i, 0)),
        out_shape=jax.ShapeDtypeStruct((n, H), jnp.float32),
    )(*sums, *ics, x_dst, Wl_cat, Wr_stack, bias.reshape(1, -1))


def _head_body(x_ref, w1_ref, b1_ref, w2_ref, o_ref):
    h = jnp.dot(x_ref[...], w1_ref[...], preferred_element_type=jnp.float32) + b1_ref[...]
    h = 0.5 * h * (1.0 + lax.erf(h * 0.7071067811865476))
    o_ref[...] = jnp.dot(h, w2_ref[...], preferred_element_type=jnp.float32)


def _head(x, hW1, hb1, hW2, hb2):
    n = x.shape[0]
    w2p = jnp.pad(hW2, ((0, 0), (0, 128 - hW2.shape[1])))
    out = pl.pallas_call(
        _head_body,
        grid=(n // ROW_BLK,),
        in_specs=[
            pl.BlockSpec((ROW_BLK, H), lambda i: (i, 0)),
            pl.BlockSpec((H, 64), lambda i: (0, 0)),
            pl.BlockSpec((1, 64), lambda i: (0, 0)),
            pl.BlockSpec((64, 128), lambda i: (0, 0)),
        ],
        out_specs=pl.BlockSpec((ROW_BLK, 128), lambda i: (i, 0)),
        out_shape=jax.ShapeDtypeStruct((n, 128), jnp.float32),
    )(x, hW1, hb1.reshape(1, -1), w2p)
    return out[:, 0] + hb2[0]


def kernel(bidder_x, ei_ba, ei_ab, ei_bd, ei_db, ei_bi, ei_ib, ln_g, ln_b, proj_W, proj_b, auction_emb, device_emb, ip_emb, Wl1, bl1, Wr1, Wl2, bl2, Wr2, hW1, hb1, hW2, hb2):
    e_ba = _pad_edges(ei_ba, NPAD_BIG)
    e_ab = _pad_edges(ei_ab, NPAD_BIG)
    e_db = _pad_edges(ei_db, NPAD_BIG)
    e_ib = _pad_edges(ei_ib, NPAD_BIG)
    e_bd = _pad_edges(ei_bd, NPAD_SMALL)
    e_bi = _pad_edges(ei_bi, NPAD_SMALL)

    ic_ba, ic_ab, ic_db, ic_ib, ic_bd, ic_bi = _edge_counts(
        e_ba[1], e_ab[1], e_db[1], e_ib[1], e_bd[1], e_bi[1])

    xb = _lnproj(bidder_x, ln_g, ln_b, proj_W, proj_b, NPAD_BIG)
    xa = jnp.pad(auction_emb, ((0, NPAD_BIG - N_A), (0, 0)))
    xd = jnp.pad(device_emb, ((0, NPAD_SMALL - N_D), (0, 0)))
    xi = jnp.pad(ip_emb, ((0, NPAD_SMALL - N_I), (0, 0)))

    def layer(xb, xa, xd, xi, Wl, bl, Wr):
        s_ba, s_ab, s_db, s_ib, s_bd, s_bi = _layer_segsums(
            xb, xa, xd, xi, *e_ba, *e_ab, *e_db, *e_ib, *e_bd, *e_bi)
        Wlb = jnp.concatenate([Wl[1], Wl[3], Wl[5]], axis=0)
        Wrb = jnp.stack([Wr[1], Wr[3], Wr[5]])
        a = _combine([s_ba], [ic_ba], xa, Wl[0], Wr[0:1], bl[0])
        b = _combine([s_ab, s_db, s_ib], [ic_ab, ic_db, ic_ib], xb, Wlb, Wrb,
                     bl[1] + bl[3] + bl[5])
        d = _combine([s_bd], [ic_bd], xd, Wl[2], Wr[2:3], bl[2])
        i = _combine([s_bi], [ic_bi], xi, Wl[4], Wr[4:5], bl[4])
        return b, a, d, i

    xb, xa, xd, xi = layer(xb, xa, xd, xi, Wl1, bl1, Wr1)
    xb, xa, xd, xi = layer(xb, xa, xd, xi, Wl2, bl2, Wr2)
    return _head(xb, hW1, hb1, hW2, hb2)[:N_B]


# speedup vs baseline: 1.8646x; 1.0473x over previous
import dataclasses
import functools

import jax
import jax.numpy as jnp
from jax import lax
from jax.experimental import pallas as pl
from jax.experimental.pallas import tpu as pltpu
from jax.experimental.pallas import tpu_sc as plsc

N_B, N_A, N_D, N_I = 50000, 50000, 10000, 10000
IN_DIM, H = 128, 128
E = 500000

NPAD_BIG = 51200
NPAD_SMALL = 10240
PART_BIG = 2560
PART_SMALL = 2560
PART_MAX = PART_BIG

E_PAD = 507904
SUB_E = E_PAD // 16
N_CHUNK = SUB_E // 1024
CAP_ROWS = (SUB_E + 256) // 128 + 2

ROW_BLK = 512

_MESH = plsc.VectorSubcoreMesh(core_axis_name="c", subcore_axis_name="s")
_CP = pltpu.CompilerParams()
if "needs_layout_passes" in pltpu.CompilerParams.__dataclass_fields__:
    _CP = dataclasses.replace(_CP, needs_layout_passes=False)


def _zero_stripes(sub, part, zbuf, bufs):
    stripe = part // 16
    zbase = sub * stripe
    nfull, rem = stripe // 64, stripe % 64
    zb = [(b * 64, 64) for b in range(nfull)]
    if rem:
        zb.append((nfull * 64, rem))
    for r0, bs in zb:
        for buf in bufs:
            pltpu.sync_copy(zbuf.at[pl.ds(0, bs)], buf.at[pl.ds(zbase + r0, bs)])

    @pl.when(sub == 0)
    def _():
        for buf in bufs:
            pltpu.sync_copy(zbuf.at[pl.ds(0, 8)], buf.at[pl.ds(part, 8)])

    return zb


def _compact(lo, part, src_hbm, dst_hbm, sidx, didx, csrc, cdst, sub, with_src):
    ebase = sub * SUB_E

    def chunk_body(ch, off):
        cb = ebase + ch * 1024
        if with_src:
            pltpu.sync_copy(src_hbm.at[pl.ds(cb, 1024)], sidx)
        pltpu.sync_copy(dst_hbm.at[pl.ds(cb, 1024)], didx)

        def vbody(g, off):
            ds_, ms, pcs, tots = [], [], [], []
            for u in range(4):
                d = didx[pl.ds((g * 4 + u) * 16, 16)]
                m = (d >= lo) & (d < lo + part)
                mi = m.astype(jnp.int32)
                ds_.append(d)
                ms.append(m)
                pcs.append(plsc.cumsum(mi))
                tots.append(plsc.all_reduce_population_count(m))
            base = off
            for u in range(4):
                pos = base + pcs[u] - 1
                ri, ci = pos >> 7, pos & 127
                if with_src:
                    s = sidx[pl.ds((g * 4 + u) * 16, 16)]
                    plsc.store_scatter(csrc, [ri, ci], s, mask=ms[u])
                plsc.store_scatter(cdst, [ri, ci], ds_[u] - lo, mask=ms[u])
                base = base + tots[u]
            return base

        return lax.fori_loop(0, 16, vbody, off)

    n_sel_v = lax.fori_loop(0, N_CHUNK, chunk_body,
                            jnp.zeros((16,), jnp.int32))
    n_sel = n_sel_v[0]

    io = lax.iota(jnp.int32, 16)
    for t in range(8):
        pidx = n_sel + t * 16 + io
        ri, ci = pidx >> 7, pidx & 127
        if with_src:
            plsc.store_scatter(csrc, [ri, ci], io + t * 16)
        plsc.store_scatter(cdst, [ri, ci], part + ((io + t * 16) & 7))
    return (n_sel + 127) >> 7


def _emit_segsum(core, sub, part, parts_per_core,
                 y_hbm, src_hbm, dst_hbm, out_hbm,
                 sidx, didx, csrc, cdst, rows, zbuf, acc, gsA, gsB):
    stripe = part // 16

    @pl.loop(0, parts_per_core)
    def _(p_local):
        p = core * parts_per_core + p_local
        lo = p * part

        zb = _zero_stripes(sub, part, zbuf, [acc])
        plsc.subcore_barrier()

        nb = _compact(lo, part, src_hbm, dst_hbm, sidx, didx, csrc, cdst, sub, True)

        rA = rows.at[pl.ds(0, 128)]
        rB = rows.at[pl.ds(128, 128)]

        @pl.when(nb > 0)
        def _():
            pltpu.async_copy(y_hbm.at[csrc.at[0]], rA, gsA)

        def pair(k, carry):
            j0 = 2 * k
            j1 = j0 + 1

            @pl.when(j1 < nb)
            def _():
                pltpu.async_copy(y_hbm.at[csrc.at[j1]], rB, gsB)

            pltpu.make_async_copy(y_hbm.at[csrc.at[j0]], rA, gsA).wait()
            pltpu.sync_copy(rA, acc.at[cdst.at[j0]], add=True)

            @pl.when(j1 < nb)
            def _():
                @pl.when(j0 + 2 < nb)
                def _():
                    pltpu.async_copy(y_hbm.at[csrc.at[j0 + 2]], rA, gsA)

                pltpu.make_async_copy(y_hbm.at[csrc.at[j1]], rB, gsB).wait()
                pltpu.sync_copy(rB, acc.at[cdst.at[j1]], add=True)

            return carry

        lax.fori_loop(0, (nb + 1) >> 1, pair, jnp.int32(0))
        plsc.subcore_barrier()

        fbase = sub * stripe
        for r0, bs in zb:
            pltpu.sync_copy(acc.at[pl.ds(fbase + r0, bs)],
                            out_hbm.at[pl.ds(lo + fbase + r0, bs)])
        plsc.subcore_barrier()


def _emit_invcount(core, sub, part, parts_per_core,
                   dst_hbm, out_hbm,
                   sidx, didx, csrc, cdst, ones, zbuf, blk, acc):
    stripe = part // 16

    @pl.loop(0, parts_per_core)
    def _(p_local):
        p = core * parts_per_core + p_local
        lo = p * part

        zb = _zero_stripes(sub, part, zbuf, [acc])
        plsc.subcore_barrier()

        nb = _compact(lo, part, None, dst_hbm, sidx, didx, csrc, cdst, sub, False)

        def bbody(j, carry):
            pltpu.sync_copy(ones, acc.at[cdst.at[j]], add=True)
            return carry

        lax.fori_loop(0, nb, bbody, jnp.int32(0))
        plsc.subcore_barrier()

        fbase = sub * stripe
        for r0, bs in zb:
            pltpu.sync_copy(acc.at[pl.ds(fbase + r0, bs)], blk.at[pl.ds(0, bs)])

            @pl.loop(0, bs)
            def _(r):
                for c in range(8):
                    slc = (r, pl.ds(c * 16, 16))
                    blk[slc] = 1.0 / jnp.maximum(blk[slc], 1.0)

            pltpu.sync_copy(blk.at[pl.ds(0, bs)],
                            out_hbm.at[pl.ds(lo + fbase + r0, bs)])
        plsc.subcore_barrier()


def _layer_segsums_body(xb, xa, xd, xi,
                        sba, dba, sab, dab, sdb, ddb, sib, dib, sbd, dbd, sbi, dbi,
                        m_ba, m_ab, m_db, m_ib, m_bd, m_bi,
                        sidx, didx, csrc, cdst, rows, zbuf, acc, gsA, gsB):
    core = lax.axis_index("c")
    sub = lax.axis_index("s")

    @pl.loop(0, 64)
    def _(i):
        for c in range(8):
            zbuf[i, pl.ds(c * 16, 16)] = jnp.zeros((16,), jnp.float32)

    convs = [
        (xb, sba, dba, m_ba, PART_BIG, 10),
        (xa, sab, dab, m_ab, PART_BIG, 10),
        (xd, sdb, ddb, m_db, PART_BIG, 10),
        (xi, sib, dib, m_ib, PART_BIG, 10),
        (xb, sbd, dbd, m_bd, PART_SMALL, 2),
        (xb, sbi, dbi, m_bi, PART_SMALL, 2),
    ]
    for y, s_e, d_e, out, prt, ppc in convs:
        _emit_segsum(core, sub, prt, ppc, y, s_e, d_e, out,
                     sidx, didx, csrc, cdst, rows, zbuf, acc, gsA, gsB)


def _counts_body(dba, dab, ddb, dib, dbd, dbi,
                 ic_ba, ic_ab, ic_db, ic_ib, ic_bd, ic_bi,
                 sidx, didx, csrc, cdst, ones, zbuf, blk, acc):
    core = lax.axis_index("c")
    sub = lax.axis_index("s")

    @pl.loop(0, 64)
    def _(i):
        for c in range(8):
            zbuf[i, pl.ds(c * 16, 16)] = jnp.zeros((16,), jnp.float32)

    @pl.loop(0, 128)
    def _(i):
        for c in range(8):
            ones[i, pl.ds(c * 16, 16)] = jnp.ones((16,), jnp.float32)

    cfgs = [
        (dba, ic_ba, PART_BIG, 10),
        (dab, ic_ab, PART_BIG, 10),
        (ddb, ic_db, PART_BIG, 10),
        (dib, ic_ib, PART_BIG, 10),
        (dbd, ic_bd, PART_SMALL, 2),
        (dbi, ic_bi, PART_SMALL, 2),
    ]
    for d_e, out, prt, ppc in cfgs:
        _emit_invcount(core, sub, prt, ppc, d_e, out,
                       sidx, didx, csrc, cdst, ones, zbuf, blk, acc)


_BIG_T = jax.ShapeDtypeStruct((NPAD_BIG, H), jnp.float32)
_SMALL_T = jax.ShapeDtypeStruct((NPAD_SMALL, H), jnp.float32)

_layer_segsums = pl.kernel(
    _layer_segsums_body,
    out_type=[_BIG_T, _BIG_T, _BIG_T, _BIG_T, _SMALL_T, _SMALL_T],
    mesh=_MESH,
    compiler_params=_CP,
    scratch_types=[
        pltpu.VMEM((1024,), jnp.int32),
        pltpu.VMEM((1024,), jnp.int32),
        pltpu.VMEM((CAP_ROWS, 128), jnp.int32),
        pltpu.VMEM((CAP_ROWS, 128), jnp.int32),
        pltpu.VMEM((256, 128), jnp.float32),
        pltpu.VMEM((64, 128), jnp.float32),
        pltpu.VMEM_SHARED((PART_MAX + 8, 128), jnp.float32),
        pltpu.SemaphoreType.DMA,
        pltpu.SemaphoreType.DMA,
    ],
)

_edge_counts = pl.kernel(
    _counts_body,
    out_type=[_BIG_T, _BIG_T, _BIG_T, _BIG_T, _SMALL_T, _SMALL_T],
    mesh=_MESH,
    compiler_params=_CP,
    scratch_types=[
        pltpu.VMEM((1024,), jnp.int32),
        pltpu.VMEM((1024,), jnp.int32),
        pltpu.VMEM((CAP_ROWS, 128), jnp.int32),
        pltpu.VMEM((CAP_ROWS, 128), jnp.int32),
        pltpu.VMEM((128, 128), jnp.float32),
        pltpu.VMEM((64, 128), jnp.float32),
        pltpu.VMEM((64, 128), jnp.float32),
        pltpu.VMEM_SHARED((PART_MAX + 8, 128), jnp.float32),
    ],
)


def _pad_edges(ei, n_dst_pad):
    pad = E_PAD - E
    srcp = jnp.concatenate([ei[0], jnp.zeros((pad,), jnp.int32)])
    dstp = jnp.concatenate([ei[1], jnp.full((pad,), n_dst_pad, jnp.int32)])
    return srcp, dstp


def _lnproj_body(x_ref, g_ref, b_ref, w_ref, pb_ref, o_ref):
    x = x_ref[...]
    mu = jnp.mean(x, axis=-1, keepdims=True)
    var = jnp.mean((x - mu) ** 2, axis=-1, keepdims=True)
    xn = ((x - mu) / jnp.sqrt(var + 1e-5)) * g_ref[...] + b_ref[...]
    o_ref[...] = jnp.dot(xn, w_ref[...], preferred_element_type=jnp.float32) + pb_ref[...]


def _lnproj(x, g, b, W, pb, n_pad):
    x = jnp.pad(x, ((0, n_pad - x.shape[0]), (0, 0)))
    return pl.pallas_call(
        _lnproj_body,
        grid=(n_pad // ROW_BLK,),
        in_specs=[
            pl.BlockSpec((ROW_BLK, IN_DIM), lambda i: (i, 0)),
            pl.BlockSpec((1, IN_DIM), lambda i: (0, 0)),
            pl.BlockSpec((1, IN_DIM), lambda i: (0, 0)),
            pl.BlockSpec((IN_DIM, H), lambda i: (0, 0)),
            pl.BlockSpec((1, H), lambda i: (0, 0)),
        ],
        out_specs=pl.BlockSpec((ROW_BLK, H), lambda i: (i, 0)),
        out_shape=jax.ShapeDtypeStruct((n_pad, H), jnp.float32),
    )(x, g.reshape(1, -1), b.reshape(1, -1), W, pb.reshape(1, -1))


def _combine_body(m, refs):
    sums = refs[:m]
    ics = refs[m:2 * m]
    xd_ref, wl_ref, wr_ref, b_ref, o_ref = refs[2 * m:]
    x = jnp.concatenate(
        [s[...] * ic[...] for s, ic in zip(sums, ics)], axis=-1)
    acc = jnp.dot(x, wl_ref[...], preferred_element_type=jnp.float32)
    xd = xd_ref[...]
    for k in range(m):
        acc = acc + jnp.dot(xd, wr_ref[k], preferred_element_type=jnp.float32)
    o_ref[...] = jax.nn.relu(acc + b_ref[...])


def _combine(sums, ics, x_dst, Wl_cat, Wr_stack, bias):
    m = len(sums)
    n = x_dst.shape[0]
    body = lambda *refs: _combine_body(m, refs)
    return pl.pallas_call(
        body,
        grid=(n // ROW_BLK,),
        in_specs=[pl.BlockSpec((ROW_BLK, H), lambda i: (i, 0))
                  for _ in range(2 * m + 1)]
        + [
            pl.BlockSpec((m * H, H), lambda i: (0, 0)),
            pl.BlockSpec((m, H, H), lambda i: (0, 0, 0)),
            pl.BlockSpec((1, H), lambda i: (0, 0)),
        ],
        out_specs=pl.BlockSpec((ROW_BLK, H), lambda i: (i, 0)),
        out_shape=jax.ShapeDtypeStruct((n, H), jnp.float32),
    )(*sums, *ics, x_dst, Wl_cat, Wr_stack, bias.reshape(1, -1))


def _head_body(x_ref, w1_ref, b1_ref, w2_ref, o_ref):
    h = jnp.dot(x_ref[...], w1_ref[...], preferred_element_type=jnp.float32) + b1_ref[...]
    h = 0.5 * h * (1.0 + lax.erf(h * 0.7071067811865476))
    o_ref[...] = jnp.dot(h, w2_ref[...], preferred_element_type=jnp.float32)


def _head(x, hW1, hb1, hW2, hb2):
    n = x.shape[0]
    w2p = jnp.pad(hW2, ((0, 0), (0, 128 - hW2.shape[1])))
    out = pl.pallas_call(
        _head_body,
        grid=(n // ROW_BLK,),
        in_specs=[
            pl.BlockSpec((ROW_BLK, H), lambda i: (i, 0)),
            pl.BlockSpec((H, 64), lambda i: (0, 0)),
            pl.BlockSpec((1, 64), lambda i: (0, 0)),
            pl.BlockSpec((64, 128), lambda i: (0, 0)),
        ],
        out_specs=pl.BlockSpec((ROW_BLK, 128), lambda i: (i, 0)),
        out_shape=jax.ShapeDtypeStruct((n, 128), jnp.float32),
    )(x, hW1, hb1.reshape(1, -1), w2p)
    return out[:, 0] + hb2[0]


def kernel(bidder_x, ei_ba, ei_ab, ei_bd, ei_db, ei_bi, ei_ib, ln_g, ln_b, proj_W, proj_b, auction_emb, device_emb, ip_emb, Wl1, bl1, Wr1, Wl2, bl2, Wr2, hW1, hb1, hW2, hb2):
    e_ba = _pad_edges(ei_ba, NPAD_BIG)
    e_ab = _pad_edges(ei_ab, NPAD_BIG)
    e_db = _pad_edges(ei_db, NPAD_BIG)
    e_ib = _pad_edges(ei_ib, NPAD_BIG)
    e_bd = _pad_edges(ei_bd, NPAD_SMALL)
    e_bi = _pad_edges(ei_bi, NPAD_SMALL)

    ic_ba, ic_ab, ic_db, ic_ib, ic_bd, ic_bi = _edge_counts(
        e_ba[1], e_ab[1], e_db[1], e_ib[1], e_bd[1], e_bi[1])

    xb = _lnproj(bidder_x, ln_g, ln_b, proj_W, proj_b, NPAD_BIG)
    xa = jnp.pad(auction_emb, ((0, NPAD_BIG - N_A), (0, 0)))
    xd = jnp.pad(device_emb, ((0, NPAD_SMALL - N_D), (0, 0)))
    xi = jnp.pad(ip_emb, ((0, NPAD_SMALL - N_I), (0, 0)))

    def layer(xb, xa, xd, xi, Wl, bl, Wr):
        s_ba, s_ab, s_db, s_ib, s_bd, s_bi = _layer_segsums(
            xb, xa, xd, xi, *e_ba, *e_ab, *e_db, *e_ib, *e_bd, *e_bi)
        Wlb = jnp.concatenate([Wl[1], Wl[3], Wl[5]], axis=0)
        Wrb = jnp.stack([Wr[1], Wr[3], Wr[5]])
        a = _combine([s_ba], [ic_ba], xa, Wl[0], Wr[0:1], bl[0])
        b = _combine([s_ab, s_db, s_ib], [ic_ab, ic_db, ic_ib], xb, Wlb, Wrb,
                     bl[1] + bl[3] + bl[5])
        d = _combine([s_bd], [ic_bd], xd, Wl[2], Wr[2:3], bl[2])
        i = _combine([s_bi], [ic_bi], xi, Wl[4], Wr[4:5], bl[4])
        return b, a, d, i

    xb, xa, xd, xi = layer(xb, xa, xd, xi, Wl1, bl1, Wr1)
    xb, xa, xd, xi = layer(xb, xa, xd, xi, Wl2, bl2, Wr2)
    return _head(xb, hW1, hb1, hW2, hb2)[:N_B]
